# baseline (device time: 28565 ns/iter reference)
import jax
import jax.numpy as jnp
from jax import lax
from jax.experimental import pallas as pl
from jax.experimental.pallas import tpu as pltpu

N_DEV = 4
N_LAYERS = 3
N_CHUNKS = 8
N_HALF = 4
N_STAGES = 2


def kernel(x, Win0, Wout0, Win1, Wout1, Win2, Wout2):
    b, d = x.shape
    bc = b // N_CHUNKS

    def body(
        x_ref,
        win0_ref,
        wout0_ref,
        win1_ref,
        wout1_ref,
        win2_ref,
        wout2_ref,
        out_ref,
        send_buf,
        recv_buf,
        xs_ref,
        send_sems,
        recv_sems,
    ):
        my = lax.axis_index("i")
        partner_a = my ^ 1
        partner_b = 3 - my
        partners = ((partner_a, partner_b), (partner_b, partner_a))

        barrier_sem = pltpu.get_barrier_semaphore()
        for nbr in (partner_a, partner_b):
            pl.semaphore_signal(
                barrier_sem,
                inc=1,
                device_id=(nbr,),
                device_id_type=pl.DeviceIdType.MESH,
            )
        pl.semaphore_wait(barrier_sem, 2)

        def exchange(l, c, s, value):
            send_buf[l, c, s] = value
            rdma = pltpu.make_async_remote_copy(
                src_ref=send_buf.at[l, c, s],
                dst_ref=recv_buf.at[l, c, s],
                send_sem=send_sems.at[l, c, s],
                recv_sem=recv_sems.at[l, c, s],
                device_id=(partners[c % 2][s],),
                device_id_type=pl.DeviceIdType.MESH,
            )
            rdma.start()
            return rdma

        layers = (
            (win0_ref, wout0_ref),
            (win1_ref, wout1_ref),
            (win2_ref, wout2_ref),
        )

        pending_b = [None] * N_CHUNKS
        n_half = N_HALF
        cph = N_CHUNKS // n_half
        bh = cph * bc

        for l, (win_ref, wout_ref) in enumerate(layers):
            win = win_ref[:, :]
            wout = wout_ref[:, :]
            rdma_a = [None] * N_CHUNKS
            part = [None] * N_CHUNKS
            for g in range(n_half):
                if l == 0:
                    xh = x_ref[pl.ds(g * bh, bh), :]
                else:
                    for c in range(cph * g, cph * (g + 1)):
                        rdma, acc = pending_b[c]
                        rdma.wait()
                        xs_ref[g, pl.ds((c % cph) * bc, bc), :] = (
                            acc + recv_buf[l - 1, c, 1]
                        )
                        pending_b[c] = None
                    xh = xs_ref[g, :, :]
                h = jnp.maximum(
                    jnp.dot(xh, win, preferred_element_type=jnp.float32),
                    0.0,
                )
                ph = jnp.dot(h, wout, preferred_element_type=jnp.float32)
                for c in range(cph * g, cph * (g + 1)):
                    part[c] = ph[(c % cph) * bc : (c % cph) * bc + bc, :]
                    rdma_a[c] = exchange(l, c, 0, part[c])
                if g >= 2:
                    for c in range(cph * (g - 2), cph * (g - 1)):
                        rdma_a[c].wait()
                        acc = part[c] + recv_buf[l, c, 0]
                        pending_b[c] = (exchange(l, c, 1, acc), acc)
            for c in range(cph * (n_half - 2), N_CHUNKS):
                rdma_a[c].wait()
                acc = part[c] + recv_buf[l, c, 0]
                pending_b[c] = (exchange(l, c, 1, acc), acc)

        for c in range(N_CHUNKS):
            rdma, acc = pending_b[c]
            rdma.wait()
            out_ref[pl.ds(c * bc, bc), :] = acc + recv_buf[N_LAYERS - 1, c, 1]

    return pl.pallas_call(
        body,
        out_shape=jax.ShapeDtypeStruct((b, d), jnp.float32),
        in_specs=[pl.BlockSpec(memory_space=pltpu.VMEM)] * 7,
        out_specs=pl.BlockSpec(memory_space=pltpu.VMEM),
        scratch_shapes=[
            pltpu.VMEM((N_LAYERS, N_CHUNKS, N_STAGES, bc, d), jnp.float32),
            pltpu.VMEM((N_LAYERS, N_CHUNKS, N_STAGES, bc, d), jnp.float32),
            pltpu.VMEM((N_HALF, b // N_HALF, d), jnp.float32),
            pltpu.SemaphoreType.DMA((N_LAYERS, N_CHUNKS, N_STAGES)),
            pltpu.SemaphoreType.DMA((N_LAYERS, N_CHUNKS, N_STAGES)),
        ],
        compiler_params=pltpu.CompilerParams(collective_id=0),
    )(x, Win0, Wout0, Win1, Wout1, Win2, Wout2)


# device time: 26016 ns/iter; 1.0980x vs baseline; 1.0980x over previous
import jax
import jax.numpy as jnp
from jax import lax
from jax.experimental import pallas as pl
from jax.experimental.pallas import tpu as pltpu

N_DEV = 4
N_LAYERS = 3
N_CHUNKS = 8
N_HALF = 4
N_STAGES = 2


def kernel(x, Win0, Wout0, Win1, Wout1, Win2, Wout2):
    b, d = x.shape
    bc = b // N_CHUNKS

    def body(
        x_ref,
        win0_ref,
        wout0_ref,
        win1_ref,
        wout1_ref,
        win2_ref,
        wout2_ref,
        out_ref,
        send_buf,
        recv_buf,
        xs_ref,
        send_sems,
        recv_sems,
    ):
        my = lax.axis_index("i")
        partner_a = my ^ 1
        partner_b = 3 - my
        partners = ((partner_a, partner_b), (partner_b, partner_a))

        barrier_sem = pltpu.get_barrier_semaphore()
        for nbr in (partner_a, partner_b):
            pl.semaphore_signal(
                barrier_sem,
                inc=1,
                device_id=(nbr,),
                device_id_type=pl.DeviceIdType.MESH,
            )
        pl.semaphore_wait(barrier_sem, 2)

        def exchange(l, c, s, value):
            send_buf[l, c, s] = value
            rdma = pltpu.make_async_remote_copy(
                src_ref=send_buf.at[l, c, s],
                dst_ref=recv_buf.at[l, c, s],
                send_sem=send_sems.at[l, c, s],
                recv_sem=recv_sems.at[l, c, s],
                device_id=(partners[c % 2][s],),
                device_id_type=pl.DeviceIdType.MESH,
            )
            rdma.start()
            return rdma

        layers = (
            (win0_ref, wout0_ref),
            (win1_ref, wout1_ref),
            (win2_ref, wout2_ref),
        )

        pending_b = [None] * N_CHUNKS
        n_half = N_HALF
        cph = N_CHUNKS // n_half
        bh = cph * bc

        for l, (win_ref, wout_ref) in enumerate(layers):
            win = win_ref[:, :]
            wout = wout_ref[:, :]
            rdma_a = [None] * N_CHUNKS
            part = [None] * N_CHUNKS
            for g in range(n_half):
                if l == 0:
                    xh = x_ref[pl.ds(g * bh, bh), :]
                else:
                    for c in range(cph * g, cph * (g + 1)):
                        rdma, acc = pending_b[c]
                        rdma.wait()
                        xs_ref[g, pl.ds((c % cph) * bc, bc), :] = (
                            acc + recv_buf[l - 1, c, 1]
                        )
                        pending_b[c] = None
                    xh = xs_ref[g, :, :]
                h = jnp.maximum(
                    jnp.dot(xh, win, preferred_element_type=jnp.float32),
                    0.0,
                )
                ph = jnp.dot(h, wout, preferred_element_type=jnp.float32)
                for c in range(cph * g, cph * (g + 1)):
                    part[c] = ph[(c % cph) * bc : (c % cph) * bc + bc, :]
                    rdma_a[c] = exchange(l, c, 0, part[c])
            for c in range(N_CHUNKS):
                rdma_a[c].wait()
                acc = part[c] + recv_buf[l, c, 0]
                pending_b[c] = (exchange(l, c, 1, acc), acc)

        for c in range(N_CHUNKS):
            rdma, acc = pending_b[c]
            rdma.wait()
            out_ref[pl.ds(c * bc, bc), :] = acc + recv_buf[N_LAYERS - 1, c, 1]

    return pl.pallas_call(
        body,
        out_shape=jax.ShapeDtypeStruct((b, d), jnp.float32),
        in_specs=[pl.BlockSpec(memory_space=pltpu.VMEM)] * 7,
        out_specs=pl.BlockSpec(memory_space=pltpu.VMEM),
        scratch_shapes=[
            pltpu.VMEM((N_LAYERS, N_CHUNKS, N_STAGES, bc, d), jnp.float32),
            pltpu.VMEM((N_LAYERS, N_CHUNKS, N_STAGES, bc, d), jnp.float32),
            pltpu.VMEM((N_HALF, b // N_HALF, d), jnp.float32),
            pltpu.SemaphoreType.DMA((N_LAYERS, N_CHUNKS, N_STAGES)),
            pltpu.SemaphoreType.DMA((N_LAYERS, N_CHUNKS, N_STAGES)),
        ],
        compiler_params=pltpu.CompilerParams(collective_id=0),
    )(x, Win0, Wout0, Win1, Wout1, Win2, Wout2)


# device time: 25780 ns/iter; 1.1080x vs baseline; 1.0092x over previous
import jax
import jax.numpy as jnp
from jax import lax
from jax.experimental import pallas as pl
from jax.experimental.pallas import tpu as pltpu

N_DEV = 4
N_LAYERS = 3
N_CHUNKS = 8
N_HALF = 4
N_STAGES = 2


def kernel(x, Win0, Wout0, Win1, Wout1, Win2, Wout2):
    b, d = x.shape
    bc = b // N_CHUNKS

    def body(
        x_ref,
        win0_ref,
        wout0_ref,
        win1_ref,
        wout1_ref,
        win2_ref,
        wout2_ref,
        out_ref,
        send_buf,
        recv_buf,
        xs_ref,
        send_sems,
        recv_sems,
    ):
        my = lax.axis_index("i")
        partner_a = my ^ 1
        partner_b = 3 - my
        partners = ((partner_a, partner_b), (partner_b, partner_a))

        barrier_sem = pltpu.get_barrier_semaphore()
        for nbr in (partner_a, partner_b):
            pl.semaphore_signal(
                barrier_sem,
                inc=1,
                device_id=(nbr,),
                device_id_type=pl.DeviceIdType.MESH,
            )
        pl.semaphore_wait(barrier_sem, 2)

        def exchange(l, c, s, value):
            send_buf[l, c, s] = value
            rdma = pltpu.make_async_remote_copy(
                src_ref=send_buf.at[l, c, s],
                dst_ref=recv_buf.at[l, c, s],
                send_sem=send_sems.at[l, c, s],
                recv_sem=recv_sems.at[l, c, s],
                device_id=(partners[c % 2][s],),
                device_id_type=pl.DeviceIdType.MESH,
            )
            rdma.start()
            return rdma

        layers = (
            (win0_ref, wout0_ref),
            (win1_ref, wout1_ref),
            (win2_ref, wout2_ref),
        )

        pending_b = [None] * N_CHUNKS
        n_half = N_HALF
        cph = N_CHUNKS // n_half
        bh = cph * bc

        for l, (win_ref, wout_ref) in enumerate(layers):
            win = win_ref[:, :]
            wout = wout_ref[:, :]
            rdma_a = [None] * N_CHUNKS
            part = [None] * N_CHUNKS
            for g in range(n_half):
                if l == 0:
                    xh = x_ref[pl.ds(g * bh, bh), :]
                else:
                    xcs = []
                    for c in range(cph * g, cph * (g + 1)):
                        rdma, acc = pending_b[c]
                        rdma.wait()
                        xcs.append(acc + recv_buf[l - 1, c, 1])
                        pending_b[c] = None
                    xh = jnp.concatenate(xcs, axis=0)
                h = jnp.maximum(
                    jnp.dot(xh, win, preferred_element_type=jnp.float32),
                    0.0,
                )
                ph = jnp.dot(h, wout, preferred_element_type=jnp.float32)
                for c in range(cph * g, cph * (g + 1)):
                    part[c] = ph[(c % cph) * bc : (c % cph) * bc + bc, :]
                    rdma_a[c] = exchange(l, c, 0, part[c])
            for c in range(N_CHUNKS):
                rdma_a[c].wait()
                acc = part[c] + recv_buf[l, c, 0]
                pending_b[c] = (exchange(l, c, 1, acc), acc)

        for c in range(N_CHUNKS):
            rdma, acc = pending_b[c]
            rdma.wait()
            out_ref[pl.ds(c * bc, bc), :] = acc + recv_buf[N_LAYERS - 1, c, 1]

    return pl.pallas_call(
        body,
        out_shape=jax.ShapeDtypeStruct((b, d), jnp.float32),
        in_specs=[pl.BlockSpec(memory_space=pltpu.VMEM)] * 7,
        out_specs=pl.BlockSpec(memory_space=pltpu.VMEM),
        scratch_shapes=[
            pltpu.VMEM((N_LAYERS, N_CHUNKS, N_STAGES, bc, d), jnp.float32),
            pltpu.VMEM((N_LAYERS, N_CHUNKS, N_STAGES, bc, d), jnp.float32),
            pltpu.VMEM((N_HALF, b // N_HALF, d), jnp.float32),
            pltpu.SemaphoreType.DMA((N_LAYERS, N_CHUNKS, N_STAGES)),
            pltpu.SemaphoreType.DMA((N_LAYERS, N_CHUNKS, N_STAGES)),
        ],
        compiler_params=pltpu.CompilerParams(collective_id=0),
    )(x, Win0, Wout0, Win1, Wout1, Win2, Wout2)
